# Initial kernel scaffold; baseline (speedup 1.0000x reference)
import jax
import jax.numpy as jnp
from jax.experimental import pallas as pl


def kernel(feature, edge_index, W, b, gamma, beta):
    raise NotImplementedError("write your pallas kernel here")


# speedup vs baseline: 5.4882x; 5.4882x over previous
import functools

import jax
import jax.numpy as jnp
from jax import lax
from jax.experimental import pallas as pl
from jax.experimental.pallas import tpu as pltpu
from jax.experimental.pallas import tpu_sc as plsc

_EPS = 1e-5

_NC = 2
_NS = 16
_NW = _NC * _NS

_K = 80
_ZR = 80


def _sc_aggregate(feature, src, dst):
    n, d = feature.shape
    e = src.shape[0]
    ept = e // _NW
    nchunk = ept // _K
    nrch = n // _ZR
    nrch_per_tile = (nrch + _NS - 1) // _NS

    mesh = plsc.VectorSubcoreMesh(core_axis_name="c", subcore_axis_name="s")

    @functools.partial(
        pl.kernel,
        out_type=jax.ShapeDtypeStruct((_NC * n, d), jnp.float32),
        mesh=mesh,
        scratch_types=[
            pltpu.VMEM((_K,), jnp.int32),
            pltpu.VMEM((_K,), jnp.int32),
            pltpu.VMEM((_K, d), jnp.float32),
            pltpu.VMEM((_ZR, d), jnp.float32),
            pltpu.VMEM_SHARED((n, d), jnp.float32),
            pltpu.SemaphoreType.DMA,
        ],
    )
    def sc_kernel(feat_hbm, src_hbm, dst_hbm, out_hbm,
                  idx_s, idx_d, rows, zbuf, acc, sem):
        cid = lax.axis_index("c")
        sid = lax.axis_index("s")
        wid = cid * _NS + sid

        def zfill(i, carry):
            for j in range(d // 16):
                zbuf[i, pl.ds(j * 16, 16)] = jnp.zeros((16,), jnp.float32)
            return carry
        lax.fori_loop(0, _ZR, zfill, 0)

        def zero_chunk(t, carry):
            ch = sid + t * _NS

            @pl.when(ch < nrch)
            def _():
                pltpu.sync_copy(zbuf, acc.at[pl.ds(ch * _ZR, _ZR)])
            return carry
        lax.fori_loop(0, nrch_per_tile, zero_chunk, 0)
        plsc.subcore_barrier()

        def body(c, carry):
            base = pl.multiple_of(wid * ept + c * _K, 8)
            pltpu.sync_copy(src_hbm.at[pl.ds(base, _K)], idx_s)
            pltpu.sync_copy(dst_hbm.at[pl.ds(base, _K)], idx_d)
            pltpu.async_copy(feat_hbm.at[idx_s], rows, sem).wait()
            pltpu.sync_copy(rows, acc.at[idx_d], add=True)
            return carry
        lax.fori_loop(0, nchunk, body, 0)
        plsc.subcore_barrier()

        def write_chunk(t, carry):
            ch = sid + t * _NS

            @pl.when(ch < nrch)
            def _():
                pltpu.sync_copy(acc.at[pl.ds(ch * _ZR, _ZR)],
                                out_hbm.at[pl.ds(cid * n + ch * _ZR, _ZR)])
            return carry
        lax.fori_loop(0, nrch_per_tile, write_chunk, 0)

    return sc_kernel(feature, src, dst)


def _tc_finish(p0, p1, W, b2, g2, be2):
    n, d = p0.shape
    br = 1000
    grid = (n // br,)

    def tc_kernel(p0_ref, p1_ref, w_ref, b_ref, g_ref, be_ref, o_ref):
        agg = p0_ref[...] + p1_ref[...]
        h = lax.dot_general(agg, w_ref[...], (((1,), (1,)), ((), ())),
                            preferred_element_type=jnp.float32)
        h = h + b_ref[...]
        mean = jnp.mean(h, axis=1, keepdims=True)
        cent = h - mean
        var = jnp.mean(cent * cent, axis=1, keepdims=True)
        o_ref[...] = cent * lax.rsqrt(var + _EPS) * g_ref[...] + be_ref[...]

    return pl.pallas_call(
        tc_kernel,
        grid=grid,
        in_specs=[
            pl.BlockSpec((br, d), lambda i: (i, 0)),
            pl.BlockSpec((br, d), lambda i: (i, 0)),
            pl.BlockSpec((d, d), lambda i: (0, 0)),
            pl.BlockSpec((1, d), lambda i: (0, 0)),
            pl.BlockSpec((1, d), lambda i: (0, 0)),
            pl.BlockSpec((1, d), lambda i: (0, 0)),
        ],
        out_specs=pl.BlockSpec((br, d), lambda i: (i, 0)),
        out_shape=jax.ShapeDtypeStruct((n, d), jnp.float32),
    )(p0, p1, W, b2, g2, be2)


def kernel(feature, edge_index, W, b, gamma, beta):
    n, d = feature.shape
    src = edge_index[0]
    dst = edge_index[1]
    partials = _sc_aggregate(feature, src, dst)
    return _tc_finish(partials[:n], partials[n:], W,
                      b.reshape(1, d), gamma.reshape(1, d), beta.reshape(1, d))


# speedup vs baseline: 9.5880x; 1.7470x over previous
import functools

import jax
import jax.numpy as jnp
from jax import lax
from jax.experimental import pallas as pl
from jax.experimental.pallas import tpu as pltpu
from jax.experimental.pallas import tpu_sc as plsc

_EPS = 1e-5

_NC = 2
_NS = 16
_NW = _NC * _NS

_K = 80
_ZR = 80


def _sc_aggregate(feature, src2, dst3):
    n, d = feature.shape
    _, nchunk, _ = dst3.shape
    ept = nchunk * _K
    nrch = n // _ZR
    nrch_per_tile = (nrch + _NS - 1) // _NS

    mesh = plsc.VectorSubcoreMesh(core_axis_name="c", subcore_axis_name="s")

    @functools.partial(
        pl.kernel,
        out_type=jax.ShapeDtypeStruct((_NC * n, d), jnp.float32),
        mesh=mesh,
        scratch_types=[
            pltpu.VMEM((ept,), jnp.int32),
            pltpu.VMEM((nchunk, _K), jnp.int32),
            pltpu.VMEM((_K, d), jnp.float32),
            pltpu.VMEM((_K, d), jnp.float32),
            pltpu.VMEM_SHARED((n, d), jnp.float32),
            pltpu.SemaphoreType.DMA,
            pltpu.SemaphoreType.DMA,
            pltpu.SemaphoreType.DMA,
        ],
    )
    def sc_kernel(feat_hbm, src_hbm, dst_hbm, out_hbm,
                  src_buf, dst_buf, rows0, rows1, acc,
                  sem0, sem1, semi):
        cid = lax.axis_index("c")
        sid = lax.axis_index("s")
        wid = cid * _NS + sid

        idx_cp0 = pltpu.async_copy(src_hbm.at[wid], src_buf, semi)
        idx_cp1 = pltpu.async_copy(dst_hbm.at[wid], dst_buf, semi)

        def zfill(i, carry):
            for j in range(d // 16):
                rows0[i, pl.ds(j * 16, 16)] = jnp.zeros((16,), jnp.float32)
            return carry
        lax.fori_loop(0, _ZR, zfill, 0)

        def zero_chunk(t, carry):
            ch = sid + t * _NS

            @pl.when(ch < nrch)
            def _():
                pltpu.sync_copy(rows0.at[pl.ds(0, _ZR)], acc.at[pl.ds(ch * _ZR, _ZR)])
            return carry
        lax.fori_loop(0, nrch_per_tile, zero_chunk, 0)
        idx_cp0.wait()
        idx_cp1.wait()
        plsc.subcore_barrier()

        pltpu.async_copy(feat_hbm.at[src_buf.at[pl.ds(0, _K)]], rows0, sem0)

        def body(c, carry):
            sc = src_buf.at[pl.ds(c * _K, _K)]
            sn = src_buf.at[pl.ds((c + 1) * _K % (nchunk * _K), _K)]

            @pl.when(c % 2 == 0)
            def _():
                pltpu.make_async_copy(feat_hbm.at[sc], rows0, sem0).wait()

                @pl.when(c + 1 < nchunk)
                def _():
                    pltpu.async_copy(feat_hbm.at[sn], rows1, sem1)
                pltpu.sync_copy(rows0, acc.at[dst_buf.at[c]], add=True)

            @pl.when(c % 2 == 1)
            def _():
                pltpu.make_async_copy(feat_hbm.at[sc], rows1, sem1).wait()

                @pl.when(c + 1 < nchunk)
                def _():
                    pltpu.async_copy(feat_hbm.at[sn], rows0, sem0)
                pltpu.sync_copy(rows1, acc.at[dst_buf.at[c]], add=True)
            return carry
        lax.fori_loop(0, nchunk, body, 0)
        plsc.subcore_barrier()

        def write_chunk(t, carry):
            ch = sid + t * _NS

            @pl.when(ch < nrch)
            def _():
                pltpu.sync_copy(acc.at[pl.ds(ch * _ZR, _ZR)],
                                out_hbm.at[pl.ds(cid * n + ch * _ZR, _ZR)])
            return carry
        lax.fori_loop(0, nrch_per_tile, write_chunk, 0)

    return sc_kernel(feature, src2, dst3)


def _tc_finish(p0, p1, W, b2, g2, be2):
    n, d = p0.shape
    br = 1000
    grid = (n // br,)

    def tc_kernel(p0_ref, p1_ref, w_ref, b_ref, g_ref, be_ref, o_ref):
        agg = p0_ref[...] + p1_ref[...]
        h = lax.dot_general(agg, w_ref[...], (((1,), (1,)), ((), ())),
                            preferred_element_type=jnp.float32)
        h = h + b_ref[...]
        mean = jnp.mean(h, axis=1, keepdims=True)
        cent = h - mean
        var = jnp.mean(cent * cent, axis=1, keepdims=True)
        o_ref[...] = cent * lax.rsqrt(var + _EPS) * g_ref[...] + be_ref[...]

    return pl.pallas_call(
        tc_kernel,
        grid=grid,
        in_specs=[
            pl.BlockSpec((br, d), lambda i: (i, 0)),
            pl.BlockSpec((br, d), lambda i: (i, 0)),
            pl.BlockSpec((d, d), lambda i: (0, 0)),
            pl.BlockSpec((1, d), lambda i: (0, 0)),
            pl.BlockSpec((1, d), lambda i: (0, 0)),
            pl.BlockSpec((1, d), lambda i: (0, 0)),
        ],
        out_specs=pl.BlockSpec((br, d), lambda i: (i, 0)),
        out_shape=jax.ShapeDtypeStruct((n, d), jnp.float32),
    )(p0, p1, W, b2, g2, be2)


def kernel(feature, edge_index, W, b, gamma, beta):
    n, d = feature.shape
    e = edge_index.shape[1]
    ept = e // _NW
    nchunk = ept // _K
    src2 = edge_index[0].reshape(_NW, ept)
    dst3 = edge_index[1].reshape(_NW, nchunk, _K)
    partials = _sc_aggregate(feature, src2, dst3)
    return _tc_finish(partials[:n], partials[n:], W,
                      b.reshape(1, d), gamma.reshape(1, d), beta.reshape(1, d))


# speedup vs baseline: 9.9496x; 1.0377x over previous
import functools

import jax
import jax.numpy as jnp
from jax import lax
from jax.experimental import pallas as pl
from jax.experimental.pallas import tpu as pltpu
from jax.experimental.pallas import tpu_sc as plsc

_EPS = 1e-5

_NC = 2
_NS = 16
_NW = _NC * _NS

_K = 80
_ZR = 80


def _sc_aggregate(feature, src2, dst3):
    n, d = feature.shape
    _, nchunk, _ = dst3.shape
    ept = nchunk * _K
    nrch = n // _ZR
    nrch_per_tile = (nrch + _NS - 1) // _NS

    mesh = plsc.VectorSubcoreMesh(core_axis_name="c", subcore_axis_name="s")

    @functools.partial(
        pl.kernel,
        out_type=jax.ShapeDtypeStruct((_NC * n, d), jnp.float32),
        mesh=mesh,
        scratch_types=[
            pltpu.VMEM((ept,), jnp.int32),
            pltpu.VMEM((nchunk, _K), jnp.int32),
            pltpu.VMEM((_K, d), jnp.float32),
            pltpu.VMEM((_K, d), jnp.float32),
            pltpu.VMEM_SHARED((n, d), jnp.float32),
            pltpu.SemaphoreType.DMA,
            pltpu.SemaphoreType.DMA,
            pltpu.SemaphoreType.DMA,
        ],
    )
    def sc_kernel(feat_hbm, src_hbm, dst_hbm, out_hbm,
                  src_buf, dst_buf, rows0, rows1, acc,
                  sem0, sem1, semi):
        cid = lax.axis_index("c")
        sid = lax.axis_index("s")
        wid = cid * _NS + sid

        idx_cp0 = pltpu.async_copy(src_hbm.at[wid], src_buf, semi)
        idx_cp1 = pltpu.async_copy(dst_hbm.at[wid], dst_buf, semi)

        def zfill(i, carry):
            for j in range(d // 16):
                rows0[i, pl.ds(j * 16, 16)] = jnp.zeros((16,), jnp.float32)
            return carry
        lax.fori_loop(0, _ZR, zfill, 0)

        def zero_chunk(t, carry):
            ch = sid + t * _NS

            @pl.when(ch < nrch)
            def _():
                pltpu.sync_copy(rows0.at[pl.ds(0, _ZR)], acc.at[pl.ds(ch * _ZR, _ZR)])
            return carry
        lax.fori_loop(0, nrch_per_tile, zero_chunk, 0)
        idx_cp0.wait()
        idx_cp1.wait()
        plsc.subcore_barrier()

        pltpu.async_copy(feat_hbm.at[src_buf.at[pl.ds(0, _K)]], rows0, sem0)

        def body(c, carry):
            sc = src_buf.at[pl.ds(c * _K, _K)]
            sn = src_buf.at[pl.ds((c + 1) * _K % (nchunk * _K), _K)]

            @pl.when(c % 2 == 0)
            def _():
                pltpu.make_async_copy(feat_hbm.at[sc], rows0, sem0).wait()

                @pl.when(c + 1 < nchunk)
                def _():
                    pltpu.async_copy(feat_hbm.at[sn], rows1, sem1)
                pltpu.sync_copy(rows0, acc.at[dst_buf.at[c]], add=True)

            @pl.when(c % 2 == 1)
            def _():
                pltpu.make_async_copy(feat_hbm.at[sc], rows1, sem1).wait()

                @pl.when(c + 1 < nchunk)
                def _():
                    pltpu.async_copy(feat_hbm.at[sn], rows0, sem0)
                pltpu.sync_copy(rows1, acc.at[dst_buf.at[c]], add=True)
            return carry
        lax.fori_loop(0, nchunk, body, 0)
        plsc.subcore_barrier()

        def write_chunk(t, carry):
            ch = sid + t * _NS

            @pl.when(ch < nrch)
            def _():
                pltpu.sync_copy(acc.at[pl.ds(ch * _ZR, _ZR)],
                                out_hbm.at[pl.ds(cid * n + ch * _ZR, _ZR)])
            return carry
        lax.fori_loop(0, nrch_per_tile, write_chunk, 0)

    return sc_kernel(feature, src2, dst3)


def _tc_finish(p3, W, b2, g2, be2):
    _, n, d = p3.shape
    br = 1000
    grid = (n // br,)

    def tc_kernel(p_ref, w_ref, b_ref, g_ref, be_ref, o_ref):
        agg = p_ref[0] + p_ref[1]
        h = lax.dot_general(agg, w_ref[...], (((1,), (1,)), ((), ())),
                            preferred_element_type=jnp.float32)
        h = h + b_ref[...]
        mean = jnp.mean(h, axis=1, keepdims=True)
        cent = h - mean
        var = jnp.mean(cent * cent, axis=1, keepdims=True)
        o_ref[...] = cent * lax.rsqrt(var + _EPS) * g_ref[...] + be_ref[...]

    return pl.pallas_call(
        tc_kernel,
        grid=grid,
        in_specs=[
            pl.BlockSpec((2, br, d), lambda i: (0, i, 0)),
            pl.BlockSpec((d, d), lambda i: (0, 0)),
            pl.BlockSpec((1, d), lambda i: (0, 0)),
            pl.BlockSpec((1, d), lambda i: (0, 0)),
            pl.BlockSpec((1, d), lambda i: (0, 0)),
        ],
        out_specs=pl.BlockSpec((br, d), lambda i: (i, 0)),
        out_shape=jax.ShapeDtypeStruct((n, d), jnp.float32),
    )(p3, W, b2, g2, be2)


def kernel(feature, edge_index, W, b, gamma, beta):
    n, d = feature.shape
    e = edge_index.shape[1]
    ept = e // _NW
    nchunk = ept // _K
    src2 = edge_index[0].reshape(_NW, ept)
    dst3 = edge_index[1].reshape(_NW, nchunk, _K)
    partials = _sc_aggregate(feature, src2, dst3)
    return _tc_finish(partials.reshape(_NC, n, d), W,
                      b.reshape(1, d), gamma.reshape(1, d), beta.reshape(1, d))


# speedup vs baseline: 13.9736x; 1.4044x over previous
import functools

import jax
import jax.numpy as jnp
from jax import lax
from jax.experimental import pallas as pl
from jax.experimental.pallas import tpu as pltpu
from jax.experimental.pallas import tpu_sc as plsc

_EPS = 1e-5

_NC = 2
_NS = 16
_NW = _NC * _NS

_K = 80
_ZR = 80
_NB = 3


def _sc_aggregate(feature, src2, dst3):
    n, d = feature.shape
    _, nchunk, _ = dst3.shape
    ept = nchunk * _K
    nrch = n // _ZR
    nrch_per_tile = (nrch + _NS - 1) // _NS

    mesh = plsc.VectorSubcoreMesh(core_axis_name="c", subcore_axis_name="s")

    @functools.partial(
        pl.kernel,
        out_type=jax.ShapeDtypeStruct((_NC * n, d), jnp.float32),
        mesh=mesh,
        scratch_types=[
            pltpu.VMEM((ept,), jnp.int32),
            pltpu.VMEM((_NB, _K), jnp.int32),
            pltpu.VMEM((_NB, _K, d), jnp.float32),
            pltpu.VMEM_SHARED((n, d), jnp.float32),
            pltpu.SemaphoreType.DMA,
            [pltpu.SemaphoreType.DMA] * _NB,
            [pltpu.SemaphoreType.DMA] * _NB,
        ],
    )
    def sc_kernel(feat_hbm, src_hbm, dst_hbm, out_hbm,
                  src_buf, dstage, rows, acc, semi, gsems, dsems):
        cid = lax.axis_index("c")
        sid = lax.axis_index("s")
        wid = cid * _NS + sid

        idx_cp = pltpu.async_copy(src_hbm.at[wid], src_buf, semi)

        def zfill(i, carry):
            for j in range(d // 16):
                rows[0, i, pl.ds(j * 16, 16)] = jnp.zeros((16,), jnp.float32)
            return carry
        lax.fori_loop(0, _ZR, zfill, 0)

        def zero_chunk(t, carry):
            ch = sid + t * _NS

            @pl.when(ch < nrch)
            def _():
                pltpu.sync_copy(rows.at[0].at[pl.ds(0, _ZR)],
                                acc.at[pl.ds(ch * _ZR, _ZR)])
            return carry
        lax.fori_loop(0, nrch_per_tile, zero_chunk, 0)
        idx_cp.wait()
        plsc.subcore_barrier()

        def gather(c, r):
            pltpu.async_copy(
                feat_hbm.at[src_buf.at[pl.ds(c * _K, _K)]], rows.at[r],
                gsems[r])

        def gather_wait(c, r):
            pltpu.make_async_copy(
                feat_hbm.at[src_buf.at[pl.ds(c * _K, _K)]], rows.at[r],
                gsems[r]).wait()

        def dfetch(c, r):
            pltpu.async_copy(dst_hbm.at[wid, c], dstage.at[r], dsems[r])

        def dfetch_wait(c, r):
            pltpu.make_async_copy(dst_hbm.at[wid, c], dstage.at[r],
                                  dsems[r]).wait()

        for r in range(_NB - 1):
            gather(r, r)
            dfetch(r, r)

        def body(c, carry):
            for r in range(_NB):
                @pl.when(c % _NB == r)
                def _():
                    gather_wait(c, r)

                    @pl.when(c + _NB - 1 < nchunk)
                    def _():
                        gather(c + _NB - 1, (r + _NB - 1) % _NB)
                        dfetch(c + _NB - 1, (r + _NB - 1) % _NB)
                    dfetch_wait(c, r)
                    pltpu.sync_copy(rows.at[r], acc.at[dstage.at[r]], add=True)
            return carry
        lax.fori_loop(0, nchunk, body, 0)
        plsc.subcore_barrier()

        def write_chunk(t, carry):
            ch = sid + t * _NS

            @pl.when(ch < nrch)
            def _():
                pltpu.sync_copy(acc.at[pl.ds(ch * _ZR, _ZR)],
                                out_hbm.at[pl.ds(cid * n + ch * _ZR, _ZR)])
            return carry
        lax.fori_loop(0, nrch_per_tile, write_chunk, 0)

    return sc_kernel(feature, src2, dst3)


def _tc_finish(p3, W, b2, g2, be2):
    _, n, d = p3.shape
    br = 1000
    grid = (n // br,)

    def tc_kernel(p_ref, w_ref, b_ref, g_ref, be_ref, o_ref):
        agg = p_ref[0] + p_ref[1]
        h = lax.dot_general(agg, w_ref[...], (((1,), (1,)), ((), ())),
                            preferred_element_type=jnp.float32)
        h = h + b_ref[...]
        mean = jnp.mean(h, axis=1, keepdims=True)
        cent = h - mean
        var = jnp.mean(cent * cent, axis=1, keepdims=True)
        o_ref[...] = cent * lax.rsqrt(var + _EPS) * g_ref[...] + be_ref[...]

    return pl.pallas_call(
        tc_kernel,
        grid=grid,
        in_specs=[
            pl.BlockSpec((2, br, d), lambda i: (0, i, 0)),
            pl.BlockSpec((d, d), lambda i: (0, 0)),
            pl.BlockSpec((1, d), lambda i: (0, 0)),
            pl.BlockSpec((1, d), lambda i: (0, 0)),
            pl.BlockSpec((1, d), lambda i: (0, 0)),
        ],
        out_specs=pl.BlockSpec((br, d), lambda i: (i, 0)),
        out_shape=jax.ShapeDtypeStruct((n, d), jnp.float32),
    )(p3, W, b2, g2, be2)


def kernel(feature, edge_index, W, b, gamma, beta):
    n, d = feature.shape
    e = edge_index.shape[1]
    ept = e // _NW
    nchunk = ept // _K
    src2 = edge_index[0].reshape(_NW, ept)
    dst3 = edge_index[1].reshape(_NW, nchunk, _K)
    partials = _sc_aggregate(feature, src2, dst3)
    return _tc_finish(partials.reshape(_NC, n, d), W,
                      b.reshape(1, d), gamma.reshape(1, d), beta.reshape(1, d))


# speedup vs baseline: 15.1065x; 1.0811x over previous
import functools

import jax
import jax.numpy as jnp
from jax import lax
from jax.experimental import pallas as pl
from jax.experimental.pallas import tpu as pltpu
from jax.experimental.pallas import tpu_sc as plsc

_EPS = 1e-5

_NC = 2
_NS = 16
_NW = _NC * _NS

_K = 80
_ZR = 80
_NB = 3


def _sc_aggregate(feature, edges4):
    n, d = feature.shape
    nchunk = edges4.shape[2]
    nrch = n // _ZR
    nrch_per_tile = (nrch + _NS - 1) // _NS

    mesh = plsc.VectorSubcoreMesh(core_axis_name="c", subcore_axis_name="s")

    @functools.partial(
        pl.kernel,
        out_type=jax.ShapeDtypeStruct((_NC * n, d), jnp.float32),
        mesh=mesh,
        scratch_types=[
            pltpu.VMEM((nchunk, _K), jnp.int32),
            pltpu.VMEM((_NB, _K), jnp.int32),
            pltpu.VMEM((_NB, _K, d), jnp.float32),
            pltpu.VMEM_SHARED((n, d), jnp.float32),
            pltpu.SemaphoreType.DMA,
            [pltpu.SemaphoreType.DMA] * _NB,
            [pltpu.SemaphoreType.DMA] * _NB,
        ],
    )
    def sc_kernel(feat_hbm, edge_hbm, out_hbm,
                  src_buf, dstage, rows, acc, semi, gsems, dsems):
        cid = lax.axis_index("c")
        sid = lax.axis_index("s")
        wid = cid * _NS + sid

        idx_cp = pltpu.async_copy(edge_hbm.at[0, wid], src_buf, semi)

        def zfill(i, carry):
            for j in range(d // 16):
                rows[0, i, pl.ds(j * 16, 16)] = jnp.zeros((16,), jnp.float32)
            return carry
        lax.fori_loop(0, _ZR, zfill, 0)

        def zero_chunk(t, carry):
            ch = sid + t * _NS

            @pl.when(ch < nrch)
            def _():
                pltpu.sync_copy(rows.at[0].at[pl.ds(0, _ZR)],
                                acc.at[pl.ds(ch * _ZR, _ZR)])
            return carry
        lax.fori_loop(0, nrch_per_tile, zero_chunk, 0)
        idx_cp.wait()
        plsc.subcore_barrier()

        def gather(c, r):
            pltpu.async_copy(feat_hbm.at[src_buf.at[c]], rows.at[r], gsems[r])

        def gather_wait(c, r):
            pltpu.make_async_copy(feat_hbm.at[src_buf.at[c]], rows.at[r],
                                  gsems[r]).wait()

        def dfetch(c, r):
            pltpu.async_copy(edge_hbm.at[1, wid, c], dstage.at[r], dsems[r])

        def dfetch_wait(c, r):
            pltpu.make_async_copy(edge_hbm.at[1, wid, c], dstage.at[r],
                                  dsems[r]).wait()

        for r in range(_NB - 1):
            gather(r, r)
            dfetch(r, r)

        def body(c, carry):
            for r in range(_NB):
                @pl.when(c % _NB == r)
                def _():
                    gather_wait(c, r)

                    @pl.when(c + _NB - 1 < nchunk)
                    def _():
                        gather(c + _NB - 1, (r + _NB - 1) % _NB)
                        dfetch(c + _NB - 1, (r + _NB - 1) % _NB)
                    dfetch_wait(c, r)
                    pltpu.sync_copy(rows.at[r], acc.at[dstage.at[r]], add=True)
            return carry
        lax.fori_loop(0, nchunk, body, 0)
        plsc.subcore_barrier()

        def write_chunk(t, carry):
            ch = sid + t * _NS

            @pl.when(ch < nrch)
            def _():
                pltpu.sync_copy(acc.at[pl.ds(ch * _ZR, _ZR)],
                                out_hbm.at[pl.ds(cid * n + ch * _ZR, _ZR)])
            return carry
        lax.fori_loop(0, nrch_per_tile, write_chunk, 0)

    return sc_kernel(feature, edges4)


def _tc_finish(p3, W, b2, g2, be2):
    _, n, d = p3.shape
    br = 1000
    grid = (n // br,)

    def tc_kernel(p_ref, w_ref, b_ref, g_ref, be_ref, o_ref):
        agg = p_ref[0] + p_ref[1]
        h = lax.dot_general(agg, w_ref[...], (((1,), (1,)), ((), ())),
                            preferred_element_type=jnp.float32)
        h = h + b_ref[...]
        mean = jnp.mean(h, axis=1, keepdims=True)
        cent = h - mean
        var = jnp.mean(cent * cent, axis=1, keepdims=True)
        o_ref[...] = cent * lax.rsqrt(var + _EPS) * g_ref[...] + be_ref[...]

    return pl.pallas_call(
        tc_kernel,
        grid=grid,
        in_specs=[
            pl.BlockSpec((2, br, d), lambda i: (0, i, 0)),
            pl.BlockSpec((d, d), lambda i: (0, 0)),
            pl.BlockSpec((1, d), lambda i: (0, 0)),
            pl.BlockSpec((1, d), lambda i: (0, 0)),
            pl.BlockSpec((1, d), lambda i: (0, 0)),
        ],
        out_specs=pl.BlockSpec((br, d), lambda i: (i, 0)),
        out_shape=jax.ShapeDtypeStruct((n, d), jnp.float32),
    )(p3, W, b2, g2, be2)


def kernel(feature, edge_index, W, b, gamma, beta):
    n, d = feature.shape
    e = edge_index.shape[1]
    partials = _sc_aggregate(
        feature, edge_index.reshape(2, _NW, (e // _NW) // _K, _K))
    return _tc_finish(partials.reshape(_NC, n, d), W,
                      b.reshape(1, d), gamma.reshape(1, d), beta.reshape(1, d))
